# Initial kernel scaffold; baseline (speedup 1.0000x reference)
import jax
import jax.numpy as jnp
from jax.experimental import pallas as pl


def kernel(conv1_w9, conv1_b, conv2_w9, conv2_b, conv3_w9, conv3_b, prop_w, prop_b, vis_lin_w, vis_lin_b, layer0_wv, layer0_bv, layer0_wo, layer0_bo, layer0_g1, layer0_be1, layer0_w1, layer0_b1, layer0_w2, layer0_b2, layer0_g2, layer0_be2, layer1_wv, layer1_bv, layer1_wo, layer1_bo, layer1_g1, layer1_be1, layer1_w1, layer1_b1, layer1_w2, layer1_b2, layer1_g2, layer1_be2, head_w, head_b, state, images):
    raise NotImplementedError("write your pallas kernel here")


# speedup vs baseline: 10.6683x; 10.6683x over previous
import jax
import jax.numpy as jnp
from jax.experimental import pallas as pl
from jax.experimental.pallas import tpu as pltpu

_ACTION_DIM = 6
_HIDDEN = 128
_LN_EPS = 1e-5
_STATE_PAD = 8
_HEAD_N = 128
_NB = 16
_TILE_B = 128

_F32 = jnp.float32
_BF16 = jnp.bfloat16

_SRC1 = {0: ((6, 0, 8), (7, 0, 8), (0, 1, 9)),
         1: ((0, 0, 8), (1, 0, 8), (2, 0, 8)),
         2: ((2, 0, 8), (3, 0, 8), (4, 0, 8)),
         3: ((4, 0, 8), (5, 0, 8), (6, 0, 8))}
_SRC2 = {0: ((2, 0, 8), (3, 0, 8), (0, 1, 9)),
         1: ((0, 0, 8), (1, 0, 8), (2, 0, 8))}
_SRC3 = ((0, 0, 8), (1, 0, 8), (0, 1, 9))


def _tower_body(img_ref, wr0, wr1, wr2, b1t,
                wb20, wb21, wb22, b2t,
                wb30, wb31, wb32, b3t, o_ref):
    nb = o_ref.shape[0]
    P = [img_ref[:, q] for q in range(8)]

    def rows(ph, lo, hi):
        return ph[:, lo:hi, :].reshape(nb * (hi - lo), ph.shape[2])

    def hpad(v, front):
        pad = ((0, 0), (1, 0), (0, 0)) if front else ((0, 0), (0, 1), (0, 0))
        return jnp.pad(v, pad)

    wrs = (wr0, wr1, wr2)
    PH1 = []
    for q1 in range(4):
        acc = jnp.zeros((nb * 8, 1024), _F32)
        for ki in range(3):
            q, lo, hi = _SRC1[q1][ki]
            acc = acc + jnp.dot(rows(P[q], lo, hi), wrs[ki][...],
                                preferred_element_type=_F32)
        v = jnp.maximum(acc + b1t[...], 0.0).astype(_BF16)
        PH1.append(hpad(v.reshape(nb, 8, 1024), q1 == 0))

    wb2 = (wb20, wb21, wb22)
    zero32 = jnp.zeros((nb * 8, 32), _BF16)
    PH2 = []
    for q2 in range(2):
        lhs = []
        for ki in range(3):
            q1s, lo, hi = _SRC2[q2][ki]
            lhs.append(jnp.concatenate([zero32, rows(PH1[q1s], lo, hi)],
                                       axis=1))
        groups = []
        for g in range(4):
            acc = jnp.zeros((nb * 8, 256), _F32)
            for ki in range(3):
                acc = acc + jnp.dot(lhs[ki][:, g * 256:g * 256 + 288],
                                    wb2[ki][...],
                                    preferred_element_type=_F32)
            groups.append(acc)
        accq = jnp.concatenate(groups, axis=1)
        v = jnp.maximum(accq + b2t[...], 0.0).astype(_BF16)
        PH2.append(hpad(v.reshape(nb, 8, 1024), q2 == 0))

    wb3 = (wb30, wb31, wb32)
    zero64 = jnp.zeros((nb * 8, 64), _BF16)
    lhs3 = []
    for ki in range(3):
        q2s, lo, hi = _SRC3[ki]
        lhs3.append(jnp.concatenate([zero64, rows(PH2[q2s], lo, hi)],
                                    axis=1))
    groups = []
    for g in range(2):
        acc = jnp.zeros((nb * 8, 512), _F32)
        for ki in range(3):
            acc = acc + jnp.dot(lhs3[ki][:, g * 512:g * 512 + 576],
                                wb3[ki][...],
                                preferred_element_type=_F32)
        groups.append(acc)
    h3 = jnp.maximum(jnp.concatenate(groups, axis=1) + b3t[...], 0.0)
    o_ref[...] = h3.astype(_BF16).reshape(nb, 8, 1024)


def _policy_body(feat_ref, state_ref, prop_w, prop_b, vlw, vlb,
                 l0_wv, l0_bv, l0_wo, l0_bo, l0_g1, l0_be1,
                 l0_w1, l0_b1, l0_w2, l0_b2, l0_g2, l0_be2,
                 l1_wv, l1_bv, l1_wo, l1_bo, l1_g1, l1_be1,
                 l1_w1, l1_b1, l1_w2, l1_b2, l1_g2, l1_be2,
                 head_w, head_b, o_ref):
    tb = o_ref.shape[0]

    def ln(x, g, b):
        mu = jnp.mean(x, axis=-1, keepdims=True)
        var = jnp.mean(jnp.square(x - mu), axis=-1, keepdims=True)
        return (x - mu) * jax.lax.rsqrt(var + _LN_EPS) * g[...] + b[...]

    vis = jnp.zeros((tb, _HIDDEN), _F32) + vlb[...]
    for u in range(8):
        vis = vis + jnp.dot(feat_ref[:, u, :], vlw[u],
                            preferred_element_type=_F32)
    prop = jnp.maximum(
        jnp.dot(state_ref[...], prop_w[...], preferred_element_type=_F32)
        + prop_b[...], 0.0)
    x = vis + prop

    for (wv, bv, wo, bo, g1, be1, w1, b1, w2, b2, g2, be2) in (
            (l0_wv, l0_bv, l0_wo, l0_bo, l0_g1, l0_be1,
             l0_w1, l0_b1, l0_w2, l0_b2, l0_g2, l0_be2),
            (l1_wv, l1_bv, l1_wo, l1_bo, l1_g1, l1_be1,
             l1_w1, l1_b1, l1_w2, l1_b2, l1_g2, l1_be2)):
        xb = x.astype(_BF16)
        v = jnp.dot(xb, wv[...], preferred_element_type=_F32) + bv[...]
        sa = jnp.dot(v.astype(_BF16), wo[...],
                     preferred_element_type=_F32) + bo[...]
        x = ln(x + sa, g1, be1)
        h = jnp.maximum(jnp.dot(x.astype(_BF16), w1[...],
                                preferred_element_type=_F32) + b1[...], 0.0)
        ff = jnp.dot(h.astype(_BF16), w2[...],
                     preferred_element_type=_F32) + b2[...]
        x = ln(x + ff, g2, be2)

    hd = jnp.dot(x.astype(_BF16), head_w[...],
                 preferred_element_type=_F32) + head_b[...]
    col = jax.lax.broadcasted_iota(jnp.int32, hd.shape, 1)
    is_std = (col >= _ACTION_DIM) & (col < 2 * _ACTION_DIM)
    o_ref[...] = jnp.where(is_std, jnp.exp(jnp.clip(hd, -20.0, 2.0)), hd)


def _banded(w9, jn, wn, cout, shift):
    cin = w9.shape[1]
    taps = []
    for kj in range(3):
        oh = jax.nn.one_hot(jnp.arange(jn) * 2 + kj + shift, wn, dtype=_F32)
        taps.append([jnp.einsum('jw,cd->wcjd', oh, w9[kj].astype(_F32))])
    acc = taps[0][0] + taps[1][0] + taps[2][0]
    return acc.reshape(wn * cin, jn * cout).astype(_BF16)


def _prep_tower_weights(c1w, c1b, c2w, c2b, c3w, c3b):
    wr = [_banded(c1w[3 * ki:3 * ki + 3], 32, 66, 32, 0) for ki in range(3)]
    wb2 = [_banded(c2w[3 * ki:3 * ki + 3], 4, 9, 64, 0) for ki in range(3)]
    wb3 = [_banded(c3w[3 * ki:3 * ki + 3], 4, 9, 128, 0) for ki in range(3)]
    b1t = jnp.tile(c1b, (1, 32))
    b2t = jnp.tile(c2b, (1, 16))
    b3t = jnp.tile(c3b, (1, 8))
    return wr + [b1t] + wb2 + [b2t] + wb3 + [b3t]


@jax.jit
def _forward(conv1_w9, conv1_b, conv2_w9, conv2_b, conv3_w9, conv3_b,
             prop_w, prop_b, vis_lin_w, vis_lin_b, layer_ws,
             head_w, head_b, state, images):
    B = state.shape[0]
    x = jnp.transpose(images, (0, 2, 3, 1)).astype(_BF16)
    xp = jnp.pad(x, ((0, 0), (1, 1), (1, 1), (0, 0)))
    xp = jnp.pad(xp, ((0, 0), (0, 6), (0, 0), (0, 0)))
    ph8 = jnp.transpose(xp.reshape(B, 9, 8, 264), (0, 2, 1, 3))

    tower_ws = _prep_tower_weights(conv1_w9, conv1_b, conv2_w9, conv2_b,
                                   conv3_w9, conv3_b)
    res = lambda shp: pl.BlockSpec(shp, lambda i: tuple(0 for _ in shp))
    feat = pl.pallas_call(
        _tower_body,
        grid=(B // _NB,),
        in_specs=[pl.BlockSpec((_NB, 8, 9, 264), lambda i: (i, 0, 0, 0))]
        + [res(w.shape) for w in tower_ws],
        out_specs=pl.BlockSpec((_NB, 8, 1024), lambda i: (i, 0, 0)),
        out_shape=jax.ShapeDtypeStruct((B, 8, 1024), _BF16),
        compiler_params=pltpu.CompilerParams(
            dimension_semantics=("parallel",),
            vmem_limit_bytes=96 * 1024 * 1024),
    )(ph8, *tower_ws)

    state_p = jnp.pad(state.astype(_BF16), ((0, 0), (0, _STATE_PAD - 6)))
    vlw8 = vis_lin_w.reshape(8, 1024, _HIDDEN)
    weights = [prop_w, prop_b, vlw8, vis_lin_b] + list(layer_ws) + \
        [head_w, head_b]
    in_specs = [pl.BlockSpec((_TILE_B, 8, 1024), lambda i: (i, 0, 0)),
                pl.BlockSpec((_TILE_B, _STATE_PAD), lambda i: (i, 0))]
    in_specs += [res(w.shape) for w in weights]

    out = pl.pallas_call(
        _policy_body,
        grid=(B // _TILE_B,),
        in_specs=in_specs,
        out_specs=pl.BlockSpec((_TILE_B, _HEAD_N), lambda i: (i, 0)),
        out_shape=jax.ShapeDtypeStruct((B, _HEAD_N), _F32),
        compiler_params=pltpu.CompilerParams(
            dimension_semantics=("parallel",),
            vmem_limit_bytes=64 * 1024 * 1024),
    )(feat, state_p, *weights)

    mean = out[:B, :_ACTION_DIM]
    std = out[:B, _ACTION_DIM:2 * _ACTION_DIM]
    value = out[:B, 2 * _ACTION_DIM:2 * _ACTION_DIM + 1]
    return mean, std, value


def kernel(conv1_w9, conv1_b, conv2_w9, conv2_b, conv3_w9, conv3_b,
           prop_w, prop_b, vis_lin_w, vis_lin_b,
           layer0_wv, layer0_bv, layer0_wo, layer0_bo, layer0_g1, layer0_be1,
           layer0_w1, layer0_b1, layer0_w2, layer0_b2, layer0_g2, layer0_be2,
           layer1_wv, layer1_bv, layer1_wo, layer1_bo, layer1_g1, layer1_be1,
           layer1_w1, layer1_b1, layer1_w2, layer1_b2, layer1_g2, layer1_be2,
           head_w, head_b, state, images):
    layer_ws = (layer0_wv, layer0_bv, layer0_wo, layer0_bo, layer0_g1,
                layer0_be1, layer0_w1, layer0_b1, layer0_w2, layer0_b2,
                layer0_g2, layer0_be2,
                layer1_wv, layer1_bv, layer1_wo, layer1_bo, layer1_g1,
                layer1_be1, layer1_w1, layer1_b1, layer1_w2, layer1_b2,
                layer1_g2, layer1_be2)
    return _forward(conv1_w9, conv1_b, conv2_w9, conv2_b, conv3_w9, conv3_b,
                    prop_w, prop_b, vis_lin_w, vis_lin_b, layer_ws,
                    head_w, head_b, state, images)


# speedup vs baseline: 11.2698x; 1.0564x over previous
import jax
import jax.numpy as jnp
from jax.experimental import pallas as pl
from jax.experimental.pallas import tpu as pltpu

_ACTION_DIM = 6
_HIDDEN = 128
_LN_EPS = 1e-5
_STATE_PAD = 8
_HEAD_N = 128
_NB = 16
_TILE_B = 128

_F32 = jnp.float32
_BF16 = jnp.bfloat16

_SRC1 = {0: ((6, 0, 8), (7, 0, 8), (0, 1, 9)),
         1: ((0, 0, 8), (1, 0, 8), (2, 0, 8)),
         2: ((2, 0, 8), (3, 0, 8), (4, 0, 8)),
         3: ((4, 0, 8), (5, 0, 8), (6, 0, 8))}
_SRC2 = {0: ((2, 0, 8), (3, 0, 8), (0, 1, 9)),
         1: ((0, 0, 8), (1, 0, 8), (2, 0, 8))}
_SRC3 = ((0, 0, 8), (1, 0, 8), (0, 1, 9))


def _tower_body(img_ref, wr0, wr1, wr2, b1t,
                wb20, wb21, wb22, b2t,
                wb30, wb31, wb32, b3t, o_ref):
    nb = o_ref.shape[0]

    def rows(ph, lo, hi):
        return ph[:, lo:hi, :].reshape(nb * (hi - lo), ph.shape[2])

    def img_rows(q, lo, hi):
        return jnp.concatenate(
            [rows(img_ref[:, c, q], lo, hi) for c in range(4)], axis=1)

    def hpad(v, front):
        pad = ((0, 0), (1, 0), (0, 0)) if front else ((0, 0), (0, 1), (0, 0))
        return jnp.pad(v, pad)

    wrs = (wr0, wr1, wr2)
    PH1 = []
    for q1 in range(4):
        acc = jnp.zeros((nb * 8, 1024), _F32)
        for ki in range(3):
            q, lo, hi = _SRC1[q1][ki]
            acc = acc + jnp.dot(img_rows(q, lo, hi), wrs[ki][...],
                                preferred_element_type=_F32)
        v = jnp.maximum(acc + b1t[...], 0.0).astype(_BF16)
        PH1.append(hpad(v.reshape(nb, 8, 1024), q1 == 0))

    wb2 = (wb20, wb21, wb22)
    zero32 = jnp.zeros((nb * 8, 32), _BF16)
    PH2 = []
    for q2 in range(2):
        lhs = []
        for ki in range(3):
            q1s, lo, hi = _SRC2[q2][ki]
            lhs.append(jnp.concatenate([zero32, rows(PH1[q1s], lo, hi)],
                                       axis=1))
        groups = []
        for g in range(4):
            acc = jnp.zeros((nb * 8, 256), _F32)
            for ki in range(3):
                acc = acc + jnp.dot(lhs[ki][:, g * 256:g * 256 + 288],
                                    wb2[ki][...],
                                    preferred_element_type=_F32)
            groups.append(acc)
        accq = jnp.concatenate(groups, axis=1)
        v = jnp.maximum(accq + b2t[...], 0.0).astype(_BF16)
        PH2.append(hpad(v.reshape(nb, 8, 1024), q2 == 0))

    wb3 = (wb30, wb31, wb32)
    zero64 = jnp.zeros((nb * 8, 64), _BF16)
    lhs3 = []
    for ki in range(3):
        q2s, lo, hi = _SRC3[ki]
        lhs3.append(jnp.concatenate([zero64, rows(PH2[q2s], lo, hi)],
                                    axis=1))
    groups = []
    for g in range(2):
        acc = jnp.zeros((nb * 8, 512), _F32)
        for ki in range(3):
            acc = acc + jnp.dot(lhs3[ki][:, g * 512:g * 512 + 576],
                                wb3[ki][...],
                                preferred_element_type=_F32)
        groups.append(acc)
    h3 = jnp.maximum(jnp.concatenate(groups, axis=1) + b3t[...], 0.0)
    o_ref[...] = h3.astype(_BF16).reshape(nb, 8, 1024)


def _policy_body(feat_ref, state_ref, prop_w, prop_b, vlw, vlb,
                 l0_wv, l0_bv, l0_wo, l0_bo, l0_g1, l0_be1,
                 l0_w1, l0_b1, l0_w2, l0_b2, l0_g2, l0_be2,
                 l1_wv, l1_bv, l1_wo, l1_bo, l1_g1, l1_be1,
                 l1_w1, l1_b1, l1_w2, l1_b2, l1_g2, l1_be2,
                 head_w, head_b, o_ref):
    tb = o_ref.shape[0]

    def ln(x, g, b):
        mu = jnp.mean(x, axis=-1, keepdims=True)
        var = jnp.mean(jnp.square(x - mu), axis=-1, keepdims=True)
        return (x - mu) * jax.lax.rsqrt(var + _LN_EPS) * g[...] + b[...]

    vis = jnp.zeros((tb, _HIDDEN), _F32) + vlb[...]
    for u in range(8):
        vis = vis + jnp.dot(feat_ref[:, u, :], vlw[u],
                            preferred_element_type=_F32)
    prop = jnp.maximum(
        jnp.dot(state_ref[...], prop_w[...], preferred_element_type=_F32)
        + prop_b[...], 0.0)
    x = vis + prop

    for (wv, bv, wo, bo, g1, be1, w1, b1, w2, b2, g2, be2) in (
            (l0_wv, l0_bv, l0_wo, l0_bo, l0_g1, l0_be1,
             l0_w1, l0_b1, l0_w2, l0_b2, l0_g2, l0_be2),
            (l1_wv, l1_bv, l1_wo, l1_bo, l1_g1, l1_be1,
             l1_w1, l1_b1, l1_w2, l1_b2, l1_g2, l1_be2)):
        xb = x.astype(_BF16)
        v = jnp.dot(xb, wv[...], preferred_element_type=_F32) + bv[...]
        sa = jnp.dot(v.astype(_BF16), wo[...],
                     preferred_element_type=_F32) + bo[...]
        x = ln(x + sa, g1, be1)
        h = jnp.maximum(jnp.dot(x.astype(_BF16), w1[...],
                                preferred_element_type=_F32) + b1[...], 0.0)
        ff = jnp.dot(h.astype(_BF16), w2[...],
                     preferred_element_type=_F32) + b2[...]
        x = ln(x + ff, g2, be2)

    hd = jnp.dot(x.astype(_BF16), head_w[...],
                 preferred_element_type=_F32) + head_b[...]
    col = jax.lax.broadcasted_iota(jnp.int32, hd.shape, 1)
    is_std = (col >= _ACTION_DIM) & (col < 2 * _ACTION_DIM)
    o_ref[...] = jnp.where(is_std, jnp.exp(jnp.clip(hd, -20.0, 2.0)), hd)


def _banded(w9, jn, wn, cout, chan_major=False):
    cin = w9.shape[1]
    pat = 'jw,cd->cwjd' if chan_major else 'jw,cd->wcjd'
    acc = 0.
    for kj in range(3):
        oh = jax.nn.one_hot(jnp.arange(jn) * 2 + kj, wn, dtype=_F32)
        acc = acc + jnp.einsum(pat, oh, w9[kj].astype(_F32))
    return acc.reshape(wn * cin, jn * cout).astype(_BF16)


def _prep_tower_weights(c1w, c1b, c2w, c2b, c3w, c3b):
    wr = [_banded(c1w[3 * ki:3 * ki + 3], 32, 66, 32, chan_major=True)
          for ki in range(3)]
    wb2 = [_banded(c2w[3 * ki:3 * ki + 3], 4, 9, 64) for ki in range(3)]
    wb3 = [_banded(c3w[3 * ki:3 * ki + 3], 4, 9, 128) for ki in range(3)]
    b1t = jnp.tile(c1b, (1, 32))
    b2t = jnp.tile(c2b, (1, 16))
    b3t = jnp.tile(c3b, (1, 8))
    return wr + [b1t] + wb2 + [b2t] + wb3 + [b3t]


@jax.jit
def _forward(conv1_w9, conv1_b, conv2_w9, conv2_b, conv3_w9, conv3_b,
             prop_w, prop_b, vis_lin_w, vis_lin_b, layer_ws,
             head_w, head_b, state, images):
    B = state.shape[0]
    xp = jnp.pad(images.astype(_BF16),
                 ((0, 0), (0, 0), (1, 7), (1, 1)))
    ph8 = jnp.transpose(xp.reshape(B, 4, 9, 8, 66), (0, 1, 3, 2, 4))

    tower_ws = _prep_tower_weights(conv1_w9, conv1_b, conv2_w9, conv2_b,
                                   conv3_w9, conv3_b)
    res = lambda shp: pl.BlockSpec(shp, lambda i: tuple(0 for _ in shp))
    feat = pl.pallas_call(
        _tower_body,
        grid=(B // _NB,),
        in_specs=[pl.BlockSpec((_NB, 4, 8, 9, 66),
                               lambda i: (i, 0, 0, 0, 0))]
        + [res(w.shape) for w in tower_ws],
        out_specs=pl.BlockSpec((_NB, 8, 1024), lambda i: (i, 0, 0)),
        out_shape=jax.ShapeDtypeStruct((B, 8, 1024), _BF16),
        compiler_params=pltpu.CompilerParams(
            dimension_semantics=("parallel",),
            vmem_limit_bytes=96 * 1024 * 1024),
    )(ph8, *tower_ws)

    state_p = jnp.pad(state.astype(_BF16), ((0, 0), (0, _STATE_PAD - 6)))
    vlw8 = vis_lin_w.reshape(8, 1024, _HIDDEN)
    weights = [prop_w, prop_b, vlw8, vis_lin_b] + list(layer_ws) + \
        [head_w, head_b]
    in_specs = [pl.BlockSpec((_TILE_B, 8, 1024), lambda i: (i, 0, 0)),
                pl.BlockSpec((_TILE_B, _STATE_PAD), lambda i: (i, 0))]
    in_specs += [res(w.shape) for w in weights]

    out = pl.pallas_call(
        _policy_body,
        grid=(B // _TILE_B,),
        in_specs=in_specs,
        out_specs=pl.BlockSpec((_TILE_B, _HEAD_N), lambda i: (i, 0)),
        out_shape=jax.ShapeDtypeStruct((B, _HEAD_N), _F32),
        compiler_params=pltpu.CompilerParams(
            dimension_semantics=("parallel",),
            vmem_limit_bytes=64 * 1024 * 1024),
    )(feat, state_p, *weights)

    mean = out[:B, :_ACTION_DIM]
    std = out[:B, _ACTION_DIM:2 * _ACTION_DIM]
    value = out[:B, 2 * _ACTION_DIM:2 * _ACTION_DIM + 1]
    return mean, std, value


def kernel(conv1_w9, conv1_b, conv2_w9, conv2_b, conv3_w9, conv3_b,
           prop_w, prop_b, vis_lin_w, vis_lin_b,
           layer0_wv, layer0_bv, layer0_wo, layer0_bo, layer0_g1, layer0_be1,
           layer0_w1, layer0_b1, layer0_w2, layer0_b2, layer0_g2, layer0_be2,
           layer1_wv, layer1_bv, layer1_wo, layer1_bo, layer1_g1, layer1_be1,
           layer1_w1, layer1_b1, layer1_w2, layer1_b2, layer1_g2, layer1_be2,
           head_w, head_b, state, images):
    layer_ws = (layer0_wv, layer0_bv, layer0_wo, layer0_bo, layer0_g1,
                layer0_be1, layer0_w1, layer0_b1, layer0_w2, layer0_b2,
                layer0_g2, layer0_be2,
                layer1_wv, layer1_bv, layer1_wo, layer1_bo, layer1_g1,
                layer1_be1, layer1_w1, layer1_b1, layer1_w2, layer1_b2,
                layer1_g2, layer1_be2)
    return _forward(conv1_w9, conv1_b, conv2_w9, conv2_b, conv3_w9, conv3_b,
                    prop_w, prop_b, vis_lin_w, vis_lin_b, layer_ws,
                    head_w, head_b, state, images)


# speedup vs baseline: 258.6037x; 22.9466x over previous
import jax
import jax.numpy as jnp
from jax.experimental import pallas as pl
from jax.experimental.pallas import tpu as pltpu

_ACTION_DIM = 6
_HIDDEN = 128
_LN_EPS = 1e-5
_STATE_PAD = 8
_HEAD_N = 128
_NB = 16
_TILE_B = 128

_F32 = jnp.float32
_BF16 = jnp.bfloat16

_SRC1 = {0: ((6, 0, 8), (7, 0, 8), (0, 1, 9)),
         1: ((0, 0, 8), (1, 0, 8), (2, 0, 8)),
         2: ((2, 0, 8), (3, 0, 8), (4, 0, 8)),
         3: ((4, 0, 8), (5, 0, 8), (6, 0, 8))}
_SRC2 = {0: ((2, 0, 8), (3, 0, 8), (0, 1, 9)),
         1: ((0, 0, 8), (1, 0, 8), (2, 0, 8))}
_SRC3 = ((0, 0, 8), (1, 0, 8), (0, 1, 9))


def _tower_body(img_ref, wr0, wr1, wr2, b1t,
                wb20, wb21, wb22, b2t,
                wb30, wb31, wb32, b3t, o_ref):
    nb = o_ref.shape[0]

    def rows(ph, lo, hi):
        return ph[:, lo:hi, :].reshape(nb * (hi - lo), ph.shape[2])

    x = img_ref[...].astype(_BF16)
    xh = jnp.pad(x, ((0, 0), (0, 0), (1, 7), (0, 0)))
    x8 = xh.reshape(nb, 4, 9, 8, 64)

    def img_rows(q, lo, hi):
        return jnp.concatenate(
            [rows(x8[:, c, :, q, :], lo, hi) for c in range(4)], axis=1)

    def hpad(v, front):
        pad = ((0, 0), (1, 0), (0, 0)) if front else ((0, 0), (0, 1), (0, 0))
        return jnp.pad(v, pad)

    wrs = (wr0, wr1, wr2)
    PH1 = []
    for q1 in range(4):
        acc = jnp.zeros((nb * 8, 1024), _F32)
        for ki in range(3):
            q, lo, hi = _SRC1[q1][ki]
            acc = acc + jnp.dot(img_rows(q, lo, hi), wrs[ki][...],
                                preferred_element_type=_F32)
        v = jnp.maximum(acc + b1t[...], 0.0).astype(_BF16)
        PH1.append(hpad(v.reshape(nb, 8, 1024), q1 == 0))

    wb2 = (wb20, wb21, wb22)
    zero32 = jnp.zeros((nb * 8, 32), _BF16)
    PH2 = []
    for q2 in range(2):
        lhs = []
        for ki in range(3):
            q1s, lo, hi = _SRC2[q2][ki]
            lhs.append(jnp.concatenate([zero32, rows(PH1[q1s], lo, hi)],
                                       axis=1))
        groups = []
        for g in range(4):
            acc = jnp.zeros((nb * 8, 256), _F32)
            for ki in range(3):
                acc = acc + jnp.dot(lhs[ki][:, g * 256:g * 256 + 288],
                                    wb2[ki][...],
                                    preferred_element_type=_F32)
            groups.append(acc)
        accq = jnp.concatenate(groups, axis=1)
        v = jnp.maximum(accq + b2t[...], 0.0).astype(_BF16)
        PH2.append(hpad(v.reshape(nb, 8, 1024), q2 == 0))

    wb3 = (wb30, wb31, wb32)
    zero64 = jnp.zeros((nb * 8, 64), _BF16)
    lhs3 = []
    for ki in range(3):
        q2s, lo, hi = _SRC3[ki]
        lhs3.append(jnp.concatenate([zero64, rows(PH2[q2s], lo, hi)],
                                    axis=1))
    groups = []
    for g in range(2):
        acc = jnp.zeros((nb * 8, 512), _F32)
        for ki in range(3):
            acc = acc + jnp.dot(lhs3[ki][:, g * 512:g * 512 + 576],
                                wb3[ki][...],
                                preferred_element_type=_F32)
        groups.append(acc)
    h3 = jnp.maximum(jnp.concatenate(groups, axis=1) + b3t[...], 0.0)
    o_ref[...] = h3.astype(_BF16).reshape(nb, 8, 1024)


def _policy_body(feat_ref, state_ref, prop_w, prop_b, vlw, vlb,
                 l0_wv, l0_bv, l0_wo, l0_bo, l0_g1, l0_be1,
                 l0_w1, l0_b1, l0_w2, l0_b2, l0_g2, l0_be2,
                 l1_wv, l1_bv, l1_wo, l1_bo, l1_g1, l1_be1,
                 l1_w1, l1_b1, l1_w2, l1_b2, l1_g2, l1_be2,
                 head_w, head_b, o_ref):
    tb = o_ref.shape[0]

    def ln(x, g, b):
        mu = jnp.mean(x, axis=-1, keepdims=True)
        var = jnp.mean(jnp.square(x - mu), axis=-1, keepdims=True)
        return (x - mu) * jax.lax.rsqrt(var + _LN_EPS) * g[...] + b[...]

    vis = jnp.zeros((tb, _HIDDEN), _F32) + vlb[...]
    for u in range(8):
        vis = vis + jnp.dot(feat_ref[:, u, :],
                            vlw[u * 1024:(u + 1) * 1024, :],
                            preferred_element_type=_F32)
    prop = jnp.maximum(
        jnp.dot(state_ref[...].astype(_BF16), prop_w[0:6, :],
                preferred_element_type=_F32)
        + prop_b[...], 0.0)
    x = vis + prop

    for (wv, bv, wo, bo, g1, be1, w1, b1, w2, b2, g2, be2) in (
            (l0_wv, l0_bv, l0_wo, l0_bo, l0_g1, l0_be1,
             l0_w1, l0_b1, l0_w2, l0_b2, l0_g2, l0_be2),
            (l1_wv, l1_bv, l1_wo, l1_bo, l1_g1, l1_be1,
             l1_w1, l1_b1, l1_w2, l1_b2, l1_g2, l1_be2)):
        xb = x.astype(_BF16)
        v = jnp.dot(xb, wv[...], preferred_element_type=_F32) + bv[...]
        sa = jnp.dot(v.astype(_BF16), wo[...],
                     preferred_element_type=_F32) + bo[...]
        x = ln(x + sa, g1, be1)
        h = jnp.maximum(jnp.dot(x.astype(_BF16), w1[...],
                                preferred_element_type=_F32) + b1[...], 0.0)
        ff = jnp.dot(h.astype(_BF16), w2[...],
                     preferred_element_type=_F32) + b2[...]
        x = ln(x + ff, g2, be2)

    hd = jnp.dot(x.astype(_BF16), head_w[...],
                 preferred_element_type=_F32) + head_b[...]
    col = jax.lax.broadcasted_iota(jnp.int32, hd.shape, 1)
    is_std = (col >= _ACTION_DIM) & (col < 2 * _ACTION_DIM)
    o_ref[...] = jnp.where(is_std, jnp.exp(jnp.clip(hd, -20.0, 2.0)), hd)


def _banded(w9, jn, wn, cout, chan_major=False, shift=0):
    cin = w9.shape[1]
    pat = 'jw,cd->cwjd' if chan_major else 'jw,cd->wcjd'
    acc = 0.
    for kj in range(3):
        oh = jax.nn.one_hot(jnp.arange(jn) * 2 + kj + shift, wn, dtype=_F32)
        acc = acc + jnp.einsum(pat, oh, w9[kj].astype(_F32))
    return acc.reshape(wn * cin, jn * cout).astype(_BF16)


def _prep_tower_weights(c1w, c1b, c2w, c2b, c3w, c3b):
    wr = [_banded(c1w[3 * ki:3 * ki + 3], 32, 64, 32, chan_major=True,
                  shift=-1) for ki in range(3)]
    wb2 = [_banded(c2w[3 * ki:3 * ki + 3], 4, 9, 64) for ki in range(3)]
    wb3 = [_banded(c3w[3 * ki:3 * ki + 3], 4, 9, 128) for ki in range(3)]
    b1t = jnp.tile(c1b, (1, 32))
    b2t = jnp.tile(c2b, (1, 16))
    b3t = jnp.tile(c3b, (1, 8))
    return wr + [b1t] + wb2 + [b2t] + wb3 + [b3t]


@jax.jit
def _forward(conv1_w9, conv1_b, conv2_w9, conv2_b, conv3_w9, conv3_b,
             prop_w, prop_b, vis_lin_w, vis_lin_b, layer_ws,
             head_w, head_b, state, images):
    B = state.shape[0]
    tower_ws = _prep_tower_weights(conv1_w9, conv1_b, conv2_w9, conv2_b,
                                   conv3_w9, conv3_b)
    res = lambda shp: pl.BlockSpec(shp, lambda i: tuple(0 for _ in shp))
    feat = pl.pallas_call(
        _tower_body,
        grid=(B // _NB,),
        in_specs=[pl.BlockSpec((_NB, 4, 64, 64), lambda i: (i, 0, 0, 0))]
        + [res(w.shape) for w in tower_ws],
        out_specs=pl.BlockSpec((_NB, 8, 1024), lambda i: (i, 0, 0)),
        out_shape=jax.ShapeDtypeStruct((B, 8, 1024), _BF16),
        compiler_params=pltpu.CompilerParams(
            dimension_semantics=("parallel",),
            vmem_limit_bytes=96 * 1024 * 1024),
    )(images, *tower_ws)

    weights = [prop_w, prop_b, vis_lin_w, vis_lin_b] + list(layer_ws) + \
        [head_w, head_b]
    in_specs = [pl.BlockSpec((_TILE_B, 8, 1024), lambda i: (i, 0, 0)),
                pl.BlockSpec((_TILE_B, 6), lambda i: (i, 0))]
    in_specs += [res(w.shape) for w in weights]

    out = pl.pallas_call(
        _policy_body,
        grid=(B // _TILE_B,),
        in_specs=in_specs,
        out_specs=pl.BlockSpec((_TILE_B, _HEAD_N), lambda i: (i, 0)),
        out_shape=jax.ShapeDtypeStruct((B, _HEAD_N), _F32),
        compiler_params=pltpu.CompilerParams(
            dimension_semantics=("parallel",),
            vmem_limit_bytes=64 * 1024 * 1024),
    )(feat, state, *weights)

    mean = out[:B, :_ACTION_DIM]
    std = out[:B, _ACTION_DIM:2 * _ACTION_DIM]
    value = out[:B, 2 * _ACTION_DIM:2 * _ACTION_DIM + 1]
    return mean, std, value


def kernel(conv1_w9, conv1_b, conv2_w9, conv2_b, conv3_w9, conv3_b,
           prop_w, prop_b, vis_lin_w, vis_lin_b,
           layer0_wv, layer0_bv, layer0_wo, layer0_bo, layer0_g1, layer0_be1,
           layer0_w1, layer0_b1, layer0_w2, layer0_b2, layer0_g2, layer0_be2,
           layer1_wv, layer1_bv, layer1_wo, layer1_bo, layer1_g1, layer1_be1,
           layer1_w1, layer1_b1, layer1_w2, layer1_b2, layer1_g2, layer1_be2,
           head_w, head_b, state, images):
    layer_ws = (layer0_wv, layer0_bv, layer0_wo, layer0_bo, layer0_g1,
                layer0_be1, layer0_w1, layer0_b1, layer0_w2, layer0_b2,
                layer0_g2, layer0_be2,
                layer1_wv, layer1_bv, layer1_wo, layer1_bo, layer1_g1,
                layer1_be1, layer1_w1, layer1_b1, layer1_w2, layer1_b2,
                layer1_g2, layer1_be2)
    return _forward(conv1_w9, conv1_b, conv2_w9, conv2_b, conv3_w9, conv3_b,
                    prop_w, prop_b, vis_lin_w, vis_lin_b, layer_ws,
                    head_w, head_b, state, images)


# speedup vs baseline: 265.9038x; 1.0282x over previous
import jax
import jax.numpy as jnp
from jax.experimental import pallas as pl
from jax.experimental.pallas import tpu as pltpu

_ACTION_DIM = 6
_HIDDEN = 128
_LN_EPS = 1e-5
_STATE_PAD = 8
_HEAD_N = 128
_NB = 32
_TILE_B = 128

_F32 = jnp.float32
_BF16 = jnp.bfloat16

_SRC1 = {0: ((6, 0, 8), (7, 0, 8), (0, 1, 9)),
         1: ((0, 0, 8), (1, 0, 8), (2, 0, 8)),
         2: ((2, 0, 8), (3, 0, 8), (4, 0, 8)),
         3: ((4, 0, 8), (5, 0, 8), (6, 0, 8))}
_SRC2 = {0: ((2, 0, 8), (3, 0, 8), (0, 1, 9)),
         1: ((0, 0, 8), (1, 0, 8), (2, 0, 8))}
_SRC3 = ((0, 0, 8), (1, 0, 8), (0, 1, 9))


def _tower_body(img_ref, wr0, wr1, wr2, b1t,
                wb20, wb21, wb22, b2t,
                wb30, wb31, wb32, b3t, o_ref):
    nb = o_ref.shape[0]

    def rows(ph, lo, hi):
        return ph[:, lo:hi, :].reshape(nb * (hi - lo), ph.shape[2])

    x = img_ref[...].astype(_BF16)
    xh = jnp.pad(x, ((0, 0), (0, 0), (1, 7), (0, 0)))
    x8 = xh.reshape(nb, 4, 9, 8, 64)

    def img_rows(q, lo, hi):
        return jnp.concatenate(
            [rows(x8[:, c, :, q, :], lo, hi) for c in range(4)], axis=1)

    def hpad(v, front):
        pad = ((0, 0), (1, 0), (0, 0)) if front else ((0, 0), (0, 1), (0, 0))
        return jnp.pad(v, pad)

    wrs = (wr0, wr1, wr2)
    PH1 = []
    for q1 in range(4):
        acc = jnp.zeros((nb * 8, 1024), _F32)
        for ki in range(3):
            q, lo, hi = _SRC1[q1][ki]
            acc = acc + jnp.dot(img_rows(q, lo, hi), wrs[ki][...],
                                preferred_element_type=_F32)
        v = jnp.maximum(acc + b1t[...], 0.0).astype(_BF16)
        PH1.append(hpad(v.reshape(nb, 8, 1024), q1 == 0))

    wb2 = (wb20, wb21, wb22)
    zero32 = jnp.zeros((nb * 8, 32), _BF16)
    PH2 = []
    for q2 in range(2):
        lhs = []
        for ki in range(3):
            q1s, lo, hi = _SRC2[q2][ki]
            lhs.append(jnp.concatenate([zero32, rows(PH1[q1s], lo, hi)],
                                       axis=1))
        groups = []
        for g in range(4):
            acc = jnp.zeros((nb * 8, 256), _F32)
            for ki in range(3):
                acc = acc + jnp.dot(lhs[ki][:, g * 256:g * 256 + 288],
                                    wb2[ki][...],
                                    preferred_element_type=_F32)
            groups.append(acc)
        accq = jnp.concatenate(groups, axis=1)
        v = jnp.maximum(accq + b2t[...], 0.0).astype(_BF16)
        PH2.append(hpad(v.reshape(nb, 8, 1024), q2 == 0))

    wb3 = (wb30, wb31, wb32)
    zero64 = jnp.zeros((nb * 8, 64), _BF16)
    lhs3 = []
    for ki in range(3):
        q2s, lo, hi = _SRC3[ki]
        lhs3.append(jnp.concatenate([zero64, rows(PH2[q2s], lo, hi)],
                                    axis=1))
    groups = []
    for g in range(2):
        acc = jnp.zeros((nb * 8, 512), _F32)
        for ki in range(3):
            acc = acc + jnp.dot(lhs3[ki][:, g * 512:g * 512 + 576],
                                wb3[ki][...],
                                preferred_element_type=_F32)
        groups.append(acc)
    h3 = jnp.maximum(jnp.concatenate(groups, axis=1) + b3t[...], 0.0)
    o_ref[...] = h3.astype(_BF16).reshape(nb, 8, 1024)


def _policy_body(feat_ref, state_ref, prop_w, prop_b, vlw, vlb,
                 l0_wv, l0_bv, l0_wo, l0_bo, l0_g1, l0_be1,
                 l0_w1, l0_b1, l0_w2, l0_b2, l0_g2, l0_be2,
                 l1_wv, l1_bv, l1_wo, l1_bo, l1_g1, l1_be1,
                 l1_w1, l1_b1, l1_w2, l1_b2, l1_g2, l1_be2,
                 head_w, head_b, o_mean, o_std, o_val):
    tb = o_mean.shape[0]

    def ln(x, g, b):
        mu = jnp.mean(x, axis=-1, keepdims=True)
        var = jnp.mean(jnp.square(x - mu), axis=-1, keepdims=True)
        return (x - mu) * jax.lax.rsqrt(var + _LN_EPS) * g[...] + b[...]

    vis = jnp.zeros((tb, _HIDDEN), _F32) + vlb[...]
    for u in range(8):
        vis = vis + jnp.dot(feat_ref[:, u, :],
                            vlw[u * 1024:(u + 1) * 1024, :],
                            preferred_element_type=_F32)
    prop = jnp.maximum(
        jnp.dot(state_ref[...].astype(_BF16), prop_w[0:6, :],
                preferred_element_type=_F32)
        + prop_b[...], 0.0)
    x = vis + prop

    for (wv, bv, wo, bo, g1, be1, w1, b1, w2, b2, g2, be2) in (
            (l0_wv, l0_bv, l0_wo, l0_bo, l0_g1, l0_be1,
             l0_w1, l0_b1, l0_w2, l0_b2, l0_g2, l0_be2),
            (l1_wv, l1_bv, l1_wo, l1_bo, l1_g1, l1_be1,
             l1_w1, l1_b1, l1_w2, l1_b2, l1_g2, l1_be2)):
        xb = x.astype(_BF16)
        v = jnp.dot(xb, wv[...], preferred_element_type=_F32) + bv[...]
        sa = jnp.dot(v.astype(_BF16), wo[...],
                     preferred_element_type=_F32) + bo[...]
        x = ln(x + sa, g1, be1)
        h = jnp.maximum(jnp.dot(x.astype(_BF16), w1[...],
                                preferred_element_type=_F32) + b1[...], 0.0)
        ff = jnp.dot(h.astype(_BF16), w2[...],
                     preferred_element_type=_F32) + b2[...]
        x = ln(x + ff, g2, be2)

    hd = jnp.dot(x.astype(_BF16), head_w[...],
                 preferred_element_type=_F32) + head_b[...]
    o_mean[...] = hd[:, :_ACTION_DIM]
    o_std[...] = jnp.exp(jnp.clip(hd[:, _ACTION_DIM:2 * _ACTION_DIM],
                                  -20.0, 2.0))
    o_val[...] = hd[:, 2 * _ACTION_DIM:2 * _ACTION_DIM + 1]


def _banded(w9, jn, wn, cout, chan_major=False, shift=0):
    cin = w9.shape[1]
    iw = jnp.arange(wn)[:, None]
    ij = jnp.arange(jn)[None, :]
    acc = jnp.zeros((cin, wn, jn, cout) if chan_major else
                    (wn, cin, jn, cout), _BF16)
    for kj in range(3):
        m = (iw == 2 * ij + kj + shift)
        tap = w9[kj]
        if chan_major:
            acc = acc + jnp.where(m[None, :, :, None],
                                  tap[:, None, None, :], 0)
        else:
            acc = acc + jnp.where(m[:, None, :, None],
                                  tap[None, :, None, :], 0)
    return acc.reshape(wn * cin, jn * cout)


def _prep_tower_weights(c1w, c1b, c2w, c2b, c3w, c3b):
    wr = [_banded(c1w[3 * ki:3 * ki + 3], 32, 64, 32, chan_major=True,
                  shift=-1) for ki in range(3)]
    wb2 = [_banded(c2w[3 * ki:3 * ki + 3], 4, 9, 64) for ki in range(3)]
    wb3 = [_banded(c3w[3 * ki:3 * ki + 3], 4, 9, 128) for ki in range(3)]
    b1t = jnp.tile(c1b, (1, 32))
    b2t = jnp.tile(c2b, (1, 16))
    b3t = jnp.tile(c3b, (1, 8))
    return wr + [b1t] + wb2 + [b2t] + wb3 + [b3t]


@jax.jit
def _forward(conv1_w9, conv1_b, conv2_w9, conv2_b, conv3_w9, conv3_b,
             prop_w, prop_b, vis_lin_w, vis_lin_b, layer_ws,
             head_w, head_b, state, images):
    B = state.shape[0]
    tower_ws = _prep_tower_weights(conv1_w9, conv1_b, conv2_w9, conv2_b,
                                   conv3_w9, conv3_b)
    res = lambda shp: pl.BlockSpec(shp, lambda i: tuple(0 for _ in shp))
    feat = pl.pallas_call(
        _tower_body,
        grid=(B // _NB,),
        in_specs=[pl.BlockSpec((_NB, 4, 64, 64), lambda i: (i, 0, 0, 0))]
        + [res(w.shape) for w in tower_ws],
        out_specs=pl.BlockSpec((_NB, 8, 1024), lambda i: (i, 0, 0)),
        out_shape=jax.ShapeDtypeStruct((B, 8, 1024), _BF16),
        compiler_params=pltpu.CompilerParams(
            dimension_semantics=("parallel",),
            vmem_limit_bytes=96 * 1024 * 1024),
    )(images, *tower_ws)

    weights = [prop_w, prop_b, vis_lin_w, vis_lin_b] + list(layer_ws) + \
        [head_w, head_b]
    in_specs = [pl.BlockSpec((_TILE_B, 8, 1024), lambda i: (i, 0, 0)),
                pl.BlockSpec((_TILE_B, 6), lambda i: (i, 0))]
    in_specs += [res(w.shape) for w in weights]

    mean, std, value = pl.pallas_call(
        _policy_body,
        grid=(B // _TILE_B,),
        in_specs=in_specs,
        out_specs=[pl.BlockSpec((_TILE_B, _ACTION_DIM), lambda i: (i, 0)),
                   pl.BlockSpec((_TILE_B, _ACTION_DIM), lambda i: (i, 0)),
                   pl.BlockSpec((_TILE_B, 1), lambda i: (i, 0))],
        out_shape=[jax.ShapeDtypeStruct((B, _ACTION_DIM), _F32),
                   jax.ShapeDtypeStruct((B, _ACTION_DIM), _F32),
                   jax.ShapeDtypeStruct((B, 1), _F32)],
        compiler_params=pltpu.CompilerParams(
            dimension_semantics=("parallel",),
            vmem_limit_bytes=64 * 1024 * 1024),
    )(feat, state, *weights)
    return mean, std, value


def kernel(conv1_w9, conv1_b, conv2_w9, conv2_b, conv3_w9, conv3_b,
           prop_w, prop_b, vis_lin_w, vis_lin_b,
           layer0_wv, layer0_bv, layer0_wo, layer0_bo, layer0_g1, layer0_be1,
           layer0_w1, layer0_b1, layer0_w2, layer0_b2, layer0_g2, layer0_be2,
           layer1_wv, layer1_bv, layer1_wo, layer1_bo, layer1_g1, layer1_be1,
           layer1_w1, layer1_b1, layer1_w2, layer1_b2, layer1_g2, layer1_be2,
           head_w, head_b, state, images):
    layer_ws = (layer0_wv, layer0_bv, layer0_wo, layer0_bo, layer0_g1,
                layer0_be1, layer0_w1, layer0_b1, layer0_w2, layer0_b2,
                layer0_g2, layer0_be2,
                layer1_wv, layer1_bv, layer1_wo, layer1_bo, layer1_g1,
                layer1_be1, layer1_w1, layer1_b1, layer1_w2, layer1_b2,
                layer1_g2, layer1_be2)
    return _forward(conv1_w9, conv1_b, conv2_w9, conv2_b, conv3_w9, conv3_b,
                    prop_w, prop_b, vis_lin_w, vis_lin_b, layer_ws,
                    head_w, head_b, state, images)
